# baseline (speedup 1.0000x reference)
import functools
import jax
import jax.numpy as jnp
import numpy as np
from jax.experimental import pallas as pl


_HID = 64


def _dense_grid(x, w, b, br=2000):
    R, K = x.shape
    C = w.shape[1]
    nb = R // br
    assert nb * br == R

    def body(x_ref, w_ref, b_ref, o_ref):
        o_ref[...] = (
            jnp.dot(x_ref[...], w_ref[...], preferred_element_type=jnp.float32)
            + b_ref[...]
        )

    return pl.pallas_call(
        body,
        grid=(nb,),
        in_specs=[
            pl.BlockSpec((br, K), lambda i: (i, 0)),
            pl.BlockSpec((K, C), lambda i: (0, 0)),
            pl.BlockSpec((1, C), lambda i: (0, 0)),
        ],
        out_specs=pl.BlockSpec((br, C), lambda i: (i, 0)),
        out_shape=jax.ShapeDtypeStruct((R, C), jnp.float32),
    )(x, w, b.reshape(1, C))


def _edge_compute(qag, kag, vmg, ea, we1, AB, w2, m2, ca, cm, be=4000):
    E = qag.shape[0]
    nb = E // be
    assert nb * be == E

    def body(q_ref, k_ref, v_ref, ea_ref, we1_ref, ab_ref, w2_ref, m2_ref,
             ca_ref, cm_ref, sc_ref, mg_ref):
        u1 = jax.nn.relu(
            jnp.dot(ea_ref[...], we1_ref[...], preferred_element_type=jnp.float32)
        )
        ef2 = jnp.dot(u1, ab_ref[...], preferred_element_type=jnp.float32)
        efa = ef2[:, :_HID]
        efm = ef2[:, _HID:]
        u = jax.nn.relu(q_ref[...] + k_ref[...] + efa + ca_ref[...])
        sc_ref[...] = jnp.dot(u, w2_ref[...], preferred_element_type=jnp.float32)
        s = v_ref[...] + efm + cm_ref[...]
        mg_ref[...] = jnp.dot(jax.nn.silu(s), m2_ref[...],
                              preferred_element_type=jnp.float32)

    return pl.pallas_call(
        body,
        grid=(nb,),
        in_specs=[
            pl.BlockSpec((be, _HID), lambda i: (i, 0)),
            pl.BlockSpec((be, _HID), lambda i: (i, 0)),
            pl.BlockSpec((be, _HID), lambda i: (i, 0)),
            pl.BlockSpec((be, 2), lambda i: (i, 0)),
            pl.BlockSpec((2, _HID), lambda i: (0, 0)),
            pl.BlockSpec((_HID, 2 * _HID), lambda i: (0, 0)),
            pl.BlockSpec((_HID, 1), lambda i: (0, 0)),
            pl.BlockSpec((_HID, _HID), lambda i: (0, 0)),
            pl.BlockSpec((1, _HID), lambda i: (0, 0)),
            pl.BlockSpec((1, _HID), lambda i: (0, 0)),
        ],
        out_specs=[
            pl.BlockSpec((be, 1), lambda i: (i, 0)),
            pl.BlockSpec((be, _HID), lambda i: (i, 0)),
        ],
        out_shape=[
            jax.ShapeDtypeStruct((E, 1), jnp.float32),
            jax.ShapeDtypeStruct((E, _HID), jnp.float32),
        ],
    )(qag, kag, vmg, ea, we1, AB, w2, m2, ca.reshape(1, -1), cm.reshape(1, -1))


def _segment_softmax(scores, seg, num_segments):
    m = jax.ops.segment_max(scores, seg, num_segments=num_segments)
    m = jnp.where(jnp.isfinite(m), m, 0.0)
    e = jnp.exp(scores - m[seg])
    s = jax.ops.segment_sum(e, seg, num_segments=num_segments)
    return e / (s[seg] + 1e-16)


def kernel(x, pos, graph_features, batch, edge_index, edge_attr, params):
    N = x.shape[0]
    E = edge_index.shape[1]
    row = edge_index[0]
    col = edge_index[1]

    Wn, bn = params['node_enc']
    Wg, bg = params['graph_enc']
    g = graph_features @ Wg + bg
    h = _dense_grid(x, Wn, bn) + g[batch]

    ea2 = jnp.concatenate([edge_attr, jnp.ones_like(edge_attr)], axis=1)

    for p in params['blocks']:
        Wq, bq = p['q']
        Wk, bk = p['k']
        Wv, bv = p['v']
        W1, b1 = p['edge1']
        W2, b2e = p['edge2']
        Wa, ba = p['att1']
        w2, b2s = p['att2']
        Wm1, bm1 = p['msg1']
        M2, b2m = p['msg2']

        Wa_q, Wa_k, Wa_e = Wa[:_HID], Wa[_HID:2 * _HID], Wa[2 * _HID:]
        Wm_v, Wm_e = Wm1[:_HID], Wm1[_HID:]

        F = jnp.concatenate([Wq @ Wa_q, Wk @ Wa_k, Wv @ Wm_v], axis=1)
        T = _dense_grid(h, F, jnp.zeros((3 * _HID,), jnp.float32))

        we1 = jnp.stack([W1[0], b1], axis=0)
        AB = jnp.concatenate([W2 @ Wa_e, W2 @ Wm_e], axis=1)
        ca = ba + bq @ Wa_q + bk @ Wa_k + b2e @ Wa_e
        cm = bm1 + bv @ Wm_v + b2e @ Wm_e

        qag = T[:, :_HID][col]
        kag = T[:, _HID:2 * _HID][row]
        vmg = T[:, 2 * _HID:][row]

        scores, msg = _edge_compute(qag, kag, vmg, ea2, we1, AB, w2, M2, ca, cm)
        scores = scores + b2s

        w = _segment_softmax(scores, col, N)
        sw = jax.ops.segment_sum(w, col, num_segments=N)
        agg = jax.ops.segment_sum(msg * w, col, num_segments=N)
        h = h + agg + sw * b2m[None, :]

    ones = jnp.ones((N,), jnp.float32)
    counts = jax.ops.segment_sum(ones, batch, num_segments=8)
    hsum = jax.ops.segment_sum(h, batch, num_segments=8)
    h_graph = hsum / jnp.maximum(counts, 1.0)[:, None]
    preds = []
    for name in ['neff', 'Aeff', 'NL', 'Disp', 'GVD']:
        z = h_graph
        layers = params['heads'][name]
        for i, (W, b) in enumerate(layers):
            z = z @ W + b
            if i < len(layers) - 1:
                z = jax.nn.silu(z)
        preds.append(z)
    return jnp.concatenate(preds, axis=-1)


# speedup vs baseline: 2.7452x; 2.7452x over previous
import functools
import jax
import jax.numpy as jnp
import numpy as np
from jax import lax
from jax.experimental import pallas as pl
from jax.experimental.pallas import tpu as pltpu
from jax.experimental.pallas import tpu_sc as plsc


_HID = 64
_NC, _NS = 2, 16
_NW = _NC * _NS
_CH = 64
_HALF = 25000
_STAB = 25088


@functools.lru_cache(maxsize=None)
def _sc_gather2_built(N, E):
    n_chunks = E // _CH
    assert n_chunks * _CH == E
    iters = (n_chunks + _NW - 1) // _NW
    iters += iters % 2
    mesh = plsc.VectorSubcoreMesh(core_axis_name="c", subcore_axis_name="s")

    assert iters % 2 == 0
    @functools.partial(
        pl.kernel, mesh=mesh,
        out_type=[jax.ShapeDtypeStruct((E, 2 * _HID), jnp.float32)] * 2,
        scratch_types=[
            pltpu.VMEM((2, _CH), jnp.int32),
            pltpu.VMEM((2, _CH), jnp.int32),
            pltpu.VMEM((2, _CH, 2 * _HID), jnp.float32),
            pltpu.VMEM((2, _CH, 2 * _HID), jnp.float32),
            pltpu.SemaphoreType.DMA,
            pltpu.SemaphoreType.DMA,
            pltpu.SemaphoreType.DMA,
        ],
    )
    def k(tq_h, tkv_h, col_h, row_h, qag_h, kvg_h, colv, rowv, qb, kb,
          sem_g, sem_w0, sem_w1):
        wid = lax.axis_index("s") * _NC + lax.axis_index("c")
        sems = (sem_w0, sem_w1)

        def one(i, i2, slot):
            cid = i * _NW + wid

            @pl.when(cid < n_chunks)
            def _():
                off = cid * _CH

                @pl.when(i2 >= 1)
                def _():
                    pltpu.make_async_copy(
                        qag_h.at[pl.ds(0, _CH)], qb.at[slot], sems[slot]).wait()
                    pltpu.make_async_copy(
                        kvg_h.at[pl.ds(0, _CH)], kb.at[slot], sems[slot]).wait()

                pltpu.sync_copy(col_h.at[pl.ds(off, _CH)], colv.at[slot])
                pltpu.sync_copy(row_h.at[pl.ds(off, _CH)], rowv.at[slot])
                gq = pltpu.async_copy(tq_h.at[colv.at[slot]], qb.at[slot], sem_g)
                gk = pltpu.async_copy(tkv_h.at[rowv.at[slot]], kb.at[slot], sem_g)
                gq.wait()
                gk.wait()
                pltpu.async_copy(qb.at[slot], qag_h.at[pl.ds(off, _CH)],
                                 sems[slot])
                pltpu.async_copy(kb.at[slot], kvg_h.at[pl.ds(off, _CH)],
                                 sems[slot])

        def body(i2, _):
            one(i2 * 2, i2, 0)
            one(i2 * 2 + 1, i2, 1)
            return ()

        lax.fori_loop(0, iters // 2, body, ())
        for slot, last_i in ((0, iters - 2), (1, iters - 1)):
            cid = last_i * _NW + wid

            @pl.when(cid < n_chunks)
            def _():
                pltpu.make_async_copy(
                    qag_h.at[pl.ds(0, _CH)], qb.at[slot], sems[slot]).wait()
                pltpu.make_async_copy(
                    kvg_h.at[pl.ds(0, _CH)], kb.at[slot], sems[slot]).wait()

    return k


def _sc_gather2(tq, tkv, col, row):
    return _sc_gather2_built(tq.shape[0], col.shape[0])(tq, tkv, col, row)


@functools.lru_cache(maxsize=None)
def _sc_softmax_sums_built(E):
    ch = 1280
    n_chunks = E // ch
    assert n_chunks * ch == E
    iters = (n_chunks + _NS - 1) // _NS
    zslice = _STAB // _NS
    mesh = plsc.VectorSubcoreMesh(core_axis_name="c", subcore_axis_name="s")

    @functools.partial(
        pl.kernel, mesh=mesh,
        out_type=[jax.ShapeDtypeStruct((E,), jnp.float32),
                  jax.ShapeDtypeStruct((_STAB,), jnp.float32),
                  jax.ShapeDtypeStruct((_STAB,), jnp.float32)],
        scratch_types=[
            pltpu.VMEM((ch,), jnp.float32),
            pltpu.VMEM((ch,), jnp.int32),
            pltpu.VMEM((ch,), jnp.float32),
            pltpu.VMEM((ch // 128, 128), jnp.int32),
            pltpu.VMEM((16,), jnp.float32),
            pltpu.VMEM((zslice,), jnp.float32),
            pltpu.VMEM_SHARED((_STAB,), jnp.float32),
            pltpu.SemaphoreType.DMA,
        ],
    )
    def k(sc_h, col_h, mx_h, e_h, s0_h, s1_h, scv, colv, ev, idxv, mxv, zv,
          stab, sem):
        c = lax.axis_index("c")
        s = lax.axis_index("s")
        pltpu.sync_copy(mx_h, mxv)
        base = c * _HALF

        def zfill(j, _):
            zv[pl.ds(j * 16, 16)] = jnp.zeros((16,), jnp.float32)
            return ()

        lax.fori_loop(0, zslice // 16, zfill, ())
        pltpu.sync_copy(zv, stab.at[pl.ds(s * zslice, zslice)])
        plsc.subcore_barrier()

        mxvec = mxv[...]

        def body(i, _):
            cid = i * _NS + s

            @pl.when(cid < n_chunks)
            def _():
                off = cid * ch
                pltpu.sync_copy(sc_h.at[pl.ds(off, ch)], scv)
                pltpu.sync_copy(col_h.at[pl.ds(off, ch)], colv)

                def step(j, _):
                    sl = pl.ds(j * 16, 16)
                    ev[sl] = jnp.exp(scv[sl] - mxvec)
                    cc = colv[sl] - base
                    ok = (cc >= 0) & (cc < _HALF)
                    idxv[lax.div(j, 8), pl.ds(lax.rem(j, 8) * 16, 16)] = (
                        jnp.where(ok, cc, _HALF))
                    return ()

                lax.fori_loop(0, ch // 16, step, ())
                for kk in range(ch // 128):
                    pltpu.sync_copy(ev.at[pl.ds(kk * 128, 128)],
                                    stab.at[idxv.at[kk]], add=True)

                @pl.when(c == 0)
                def _():
                    pltpu.sync_copy(ev, e_h.at[pl.ds(off, ch)])

            return ()

        lax.fori_loop(0, iters, body, ())
        plsc.subcore_barrier()

        pltpu.sync_copy(stab.at[pl.ds(s * zslice, zslice)], zv)

        @pl.when(c == 0)
        def _():
            pltpu.sync_copy(zv, s0_h.at[pl.ds(s * zslice, zslice)])

        @pl.when(c == 1)
        def _():
            pltpu.sync_copy(zv, s1_h.at[pl.ds(s * zslice, zslice)])

    return k


def _sc_softmax_sums(scores, col, mx16):
    return _sc_softmax_sums_built(scores.shape[0])(scores, col, mx16)


@functools.lru_cache(maxsize=None)
def _sc_scatter_rows_built(E):
    HH = 16
    ch = 640
    n_chunks = E // ch
    assert n_chunks * ch == E
    iters = (n_chunks + _NS - 1) // _NS
    zslice = _STAB // _NS
    zrows = 112
    mesh = plsc.VectorSubcoreMesh(core_axis_name="c", subcore_axis_name="s")

    @functools.partial(
        pl.kernel, mesh=mesh,
        out_type=[jax.ShapeDtypeStruct((_STAB, HH), jnp.float32)] * 8,
        scratch_types=[
            pltpu.VMEM((ch, HH), jnp.float32),
            pltpu.VMEM((ch,), jnp.int32),
            pltpu.VMEM((ch // 128, 128), jnp.int32),
            pltpu.VMEM((zrows, HH), jnp.float32),
            pltpu.VMEM_SHARED((_STAB, HH), jnp.float32),
            pltpu.SemaphoreType.DMA,
        ],
    )
    def k(m0_h, m1_h, m2_h, m3_h, col_h, a0_h, a1_h, b0_h, b1_h,
          c0_h, c1_h, d0_h, d1_h, mv, colv, idxv, zb, atab, sem):
        c = lax.axis_index("c")
        s = lax.axis_index("s")
        base = c * _HALF

        def zrow(j, _):
            for l in range(HH // 16):
                zb[j, pl.ds(l * 16, 16)] = jnp.zeros((16,), jnp.float32)
            return ()

        lax.fori_loop(0, zrows, zrow, ())

        for msh_h, o0_h, o1_h in ((m0_h, a0_h, a1_h), (m1_h, b0_h, b1_h),
                                  (m2_h, c0_h, c1_h), (m3_h, d0_h, d1_h)):
            for t in range(zslice // zrows):
                pltpu.sync_copy(zb,
                                atab.at[pl.ds(s * zslice + t * zrows, zrows)])
            plsc.subcore_barrier()

            def body(i, _):
                cid = i * _NS + s

                @pl.when(cid < n_chunks)
                def _():
                    off = cid * ch
                    pltpu.sync_copy(msh_h.at[pl.ds(off, ch)], mv)
                    pltpu.sync_copy(col_h.at[pl.ds(off, ch)], colv)

                    def step(j, _):
                        sl = pl.ds(j * 16, 16)
                        cc = colv[sl] - base
                        ok = (cc >= 0) & (cc < _HALF)
                        idxv[lax.div(j, 8), pl.ds(lax.rem(j, 8) * 16, 16)] = (
                            jnp.where(ok, cc, _HALF))
                        return ()

                    lax.fori_loop(0, ch // 16, step, ())
                    for kk in range(ch // 128):
                        pltpu.sync_copy(mv.at[pl.ds(kk * 128, 128)],
                                        atab.at[idxv.at[kk]], add=True)

                return ()

            lax.fori_loop(0, iters, body, ())
            plsc.subcore_barrier()

            for t in range(zslice // zrows):
                r0 = s * zslice + t * zrows
                pltpu.sync_copy(atab.at[pl.ds(r0, zrows)], zb)

                @pl.when(c == 0)
                def _():
                    pltpu.sync_copy(zb, o0_h.at[pl.ds(r0, zrows)])

                @pl.when(c == 1)
                def _():
                    pltpu.sync_copy(zb, o1_h.at[pl.ds(r0, zrows)])

            lax.fori_loop(0, zrows, zrow, ())
            plsc.subcore_barrier()

    return k


def _sc_scatter_rows(m0, m1, m2, m3, col):
    return _sc_scatter_rows_built(m0.shape[0])(m0, m1, m2, m3, col)


def _enc_kernel(x, oh, wenc, benc, g, f, br=2000):
    R, K = x.shape

    def body(x_ref, oh_ref, w_ref, b_ref, g_ref, f_ref, h_ref, tq_ref, tkv_ref):
        h = (jnp.dot(x_ref[...], w_ref[...], preferred_element_type=jnp.float32)
             + b_ref[...]
             + jnp.dot(oh_ref[...], g_ref[...], preferred_element_type=jnp.float32))
        h_ref[...] = h
        t = jnp.dot(h, f_ref[...], preferred_element_type=jnp.float32)
        tq_ref[...] = jnp.concatenate(
            [t[:, :_HID], jnp.zeros_like(t[:, :_HID])], axis=1)
        tkv_ref[...] = t[:, _HID:]

    nb = R // br
    return pl.pallas_call(
        body,
        grid=(nb,),
        in_specs=[
            pl.BlockSpec((br, K), lambda i: (i, 0)),
            pl.BlockSpec((br, 8), lambda i: (i, 0)),
            pl.BlockSpec((K, _HID), lambda i: (0, 0)),
            pl.BlockSpec((1, _HID), lambda i: (0, 0)),
            pl.BlockSpec((8, _HID), lambda i: (0, 0)),
            pl.BlockSpec((_HID, 3 * _HID), lambda i: (0, 0)),
        ],
        out_specs=[
            pl.BlockSpec((br, _HID), lambda i: (i, 0)),
            pl.BlockSpec((br, 2 * _HID), lambda i: (i, 0)),
            pl.BlockSpec((br, 2 * _HID), lambda i: (i, 0)),
        ],
        out_shape=[
            jax.ShapeDtypeStruct((R, _HID), jnp.float32),
            jax.ShapeDtypeStruct((R, 2 * _HID), jnp.float32),
            jax.ShapeDtypeStruct((R, 2 * _HID), jnp.float32),
        ],
    )(x, oh, wenc, benc.reshape(1, -1), g, f)


def _upd_kernel(h, agg, s, m2, b2m, f, br=2000):
    R = h.shape[0]

    def body(h_ref, a_ref, s_ref, m2_ref, b_ref, f_ref, h_out, tq_ref, tkv_ref):
        r = 1.0 / (s_ref[...] + 1e-16)
        hn = (h_ref[...]
              + jnp.dot(a_ref[...] * r, m2_ref[...],
                        preferred_element_type=jnp.float32)
              + (s_ref[...] * r) * b_ref[...])
        h_out[...] = hn
        t = jnp.dot(hn, f_ref[...], preferred_element_type=jnp.float32)
        tq_ref[...] = jnp.concatenate(
            [t[:, :_HID], jnp.zeros_like(t[:, :_HID])], axis=1)
        tkv_ref[...] = t[:, _HID:]

    nb = R // br
    return pl.pallas_call(
        body,
        grid=(nb,),
        in_specs=[
            pl.BlockSpec((br, _HID), lambda i: (i, 0)),
            pl.BlockSpec((br, _HID), lambda i: (i, 0)),
            pl.BlockSpec((br, 1), lambda i: (i, 0)),
            pl.BlockSpec((_HID, _HID), lambda i: (0, 0)),
            pl.BlockSpec((1, _HID), lambda i: (0, 0)),
            pl.BlockSpec((_HID, 3 * _HID), lambda i: (0, 0)),
        ],
        out_specs=[
            pl.BlockSpec((br, _HID), lambda i: (i, 0)),
            pl.BlockSpec((br, 2 * _HID), lambda i: (i, 0)),
            pl.BlockSpec((br, 2 * _HID), lambda i: (i, 0)),
        ],
        out_shape=[
            jax.ShapeDtypeStruct((R, _HID), jnp.float32),
            jax.ShapeDtypeStruct((R, 2 * _HID), jnp.float32),
            jax.ShapeDtypeStruct((R, 2 * _HID), jnp.float32),
        ],
    )(h, agg, s, m2, b2m.reshape(1, -1), f)


def _fin_kernel(h, agg, s, m2, b2m, oh, br=2000):
    R = h.shape[0]
    nb = R // br

    def body(h_ref, a_ref, s_ref, m2_ref, b_ref, oh_ref, hs_ref, ct_ref):
        i = pl.program_id(0)
        r = 1.0 / (s_ref[...] + 1e-16)
        hn = (h_ref[...]
              + jnp.dot(a_ref[...] * r, m2_ref[...],
                        preferred_element_type=jnp.float32)
              + (s_ref[...] * r) * b_ref[...])
        part = lax.dot_general(oh_ref[...], hn, (((0,), (0,)), ((), ())),
                               preferred_element_type=jnp.float32)
        cpart = jnp.sum(oh_ref[...], axis=0, keepdims=True)

        @pl.when(i == 0)
        def _():
            hs_ref[...] = part
            ct_ref[...] = cpart

        @pl.when(i > 0)
        def _():
            hs_ref[...] += part
            ct_ref[...] += cpart

    return pl.pallas_call(
        body,
        grid=(nb,),
        in_specs=[
            pl.BlockSpec((br, _HID), lambda i: (i, 0)),
            pl.BlockSpec((br, _HID), lambda i: (i, 0)),
            pl.BlockSpec((br, 1), lambda i: (i, 0)),
            pl.BlockSpec((_HID, _HID), lambda i: (0, 0)),
            pl.BlockSpec((1, _HID), lambda i: (0, 0)),
            pl.BlockSpec((br, 8), lambda i: (i, 0)),
        ],
        out_specs=[
            pl.BlockSpec((8, _HID), lambda i: (0, 0)),
            pl.BlockSpec((1, 8), lambda i: (0, 0)),
        ],
        out_shape=[
            jax.ShapeDtypeStruct((8, _HID), jnp.float32),
            jax.ShapeDtypeStruct((1, 8), jnp.float32),
        ],
    )(h, agg, s, m2, b2m.reshape(1, -1), oh)


def _score_kernel(qag, kvg, ea2, we1, a, w2, ca, be=4000):
    E = qag.shape[0]
    nb = E // be

    def body(q_ref, k_ref, ea_ref, we1_ref, a_ref, w2_ref, ca_ref,
             sc_ref, mx_ref):
        i = pl.program_id(0)
        u1 = jax.nn.relu(jnp.dot(ea_ref[...], we1_ref[...],
                                 preferred_element_type=jnp.float32))
        efa = jnp.dot(u1, a_ref[...], preferred_element_type=jnp.float32)
        u = jax.nn.relu(q_ref[...][:, :_HID] + k_ref[...][:, :_HID]
                        + efa + ca_ref[...])
        sc = jnp.dot(u, w2_ref[...], preferred_element_type=jnp.float32)
        sc_ref[...] = sc
        bmax = jnp.max(sc, axis=0, keepdims=True)

        @pl.when(i == 0)
        def _():
            mx_ref[...] = bmax

        @pl.when(i > 0)
        def _():
            mx_ref[...] = jnp.maximum(mx_ref[...], bmax)

    return pl.pallas_call(
        body,
        grid=(nb,),
        in_specs=[
            pl.BlockSpec((be, 2 * _HID), lambda i: (i, 0)),
            pl.BlockSpec((be, 2 * _HID), lambda i: (i, 0)),
            pl.BlockSpec((be, 2), lambda i: (i, 0)),
            pl.BlockSpec((2, _HID), lambda i: (0, 0)),
            pl.BlockSpec((_HID, _HID), lambda i: (0, 0)),
            pl.BlockSpec((_HID, 1), lambda i: (0, 0)),
            pl.BlockSpec((1, _HID), lambda i: (0, 0)),
        ],
        out_specs=[
            pl.BlockSpec((be, 1), lambda i: (i, 0)),
            pl.BlockSpec((1, 1), lambda i: (0, 0)),
        ],
        out_shape=[
            jax.ShapeDtypeStruct((E, 1), jnp.float32),
            jax.ShapeDtypeStruct((1, 1), jnp.float32),
        ],
    )(qag, kvg, ea2, we1, a, w2, ca.reshape(1, -1))


def _msg_kernel(kvg, ea2, e, we1, b, cm, be=4000):
    E = kvg.shape[0]
    nb = E // be

    def body(v_ref, ea_ref, e_ref, we1_ref, b_ref, cm_ref,
             oa_ref, ob_ref, oc_ref, od_ref):
        u1 = jax.nn.relu(jnp.dot(ea_ref[...], we1_ref[...],
                                 preferred_element_type=jnp.float32))
        efm = jnp.dot(u1, b_ref[...], preferred_element_type=jnp.float32)
        sm = v_ref[...][:, _HID:] + efm + cm_ref[...]
        m = jax.nn.silu(sm) * e_ref[...]
        oa_ref[...] = m[:, :16]
        ob_ref[...] = m[:, 16:32]
        oc_ref[...] = m[:, 32:48]
        od_ref[...] = m[:, 48:]

    return pl.pallas_call(
        body,
        grid=(nb,),
        in_specs=[
            pl.BlockSpec((be, 2 * _HID), lambda i: (i, 0)),
            pl.BlockSpec((be, 2), lambda i: (i, 0)),
            pl.BlockSpec((be, 1), lambda i: (i, 0)),
            pl.BlockSpec((2, _HID), lambda i: (0, 0)),
            pl.BlockSpec((_HID, _HID), lambda i: (0, 0)),
            pl.BlockSpec((1, _HID), lambda i: (0, 0)),
        ],
        out_specs=[pl.BlockSpec((be, 16), lambda i: (i, 0))] * 4,
        out_shape=[jax.ShapeDtypeStruct((E, 16), jnp.float32)] * 4,
    )(kvg, ea2, e, we1, b, cm.reshape(1, -1))


def kernel(x, pos, graph_features, batch, edge_index, edge_attr, params):
    N = x.shape[0]
    E = edge_index.shape[1]
    row = edge_index[0]
    col = edge_index[1]
    oh = jax.nn.one_hot(batch, 8, dtype=jnp.float32)

    Wn, bn = params['node_enc']
    Wg, bg = params['graph_enc']
    g = graph_features @ Wg + bg

    ea2 = jnp.concatenate([edge_attr, jnp.ones_like(edge_attr)], axis=1)

    folded = []
    for p in params['blocks']:
        Wq, bq = p['q']
        Wk, bk = p['k']
        Wv, bv = p['v']
        W1, b1 = p['edge1']
        W2, b2e = p['edge2']
        Wa, ba = p['att1']
        w2, b2s = p['att2']
        Wm1, bm1 = p['msg1']
        M2, b2m = p['msg2']
        Wa_q, Wa_k, Wa_e = Wa[:_HID], Wa[_HID:2 * _HID], Wa[2 * _HID:]
        Wm_v, Wm_e = Wm1[:_HID], Wm1[_HID:]
        F = jnp.concatenate([Wq @ Wa_q, Wk @ Wa_k, Wv @ Wm_v], axis=1)
        we1 = jnp.stack([W1[0], b1], axis=0)
        A = W2 @ Wa_e
        B = W2 @ Wm_e
        ca = ba + bq @ Wa_q + bk @ Wa_k + b2e @ Wa_e
        cm = bm1 + bv @ Wm_v + b2e @ Wm_e
        folded.append(dict(F=F, we1=we1, A=A, B=B, ca=ca, cm=cm,
                           M2=M2, b2m=b2m, w2=w2))

    h, tq, tkv = _enc_kernel(x, oh, Wn, bn, g, folded[0]['F'])

    for li, fd in enumerate(folded):
        qag = tq[col]
        kvg = tkv[row]
        scores, mx = _score_kernel(qag, kvg, ea2, fd['we1'], fd['A'],
                                   fd['w2'], fd['ca'])
        e = jnp.exp(scores.reshape(E) - mx.reshape(()))
        s = jax.ops.segment_sum(e, col, num_segments=N).reshape(N, 1)
        m0, m1, m2, m3 = _msg_kernel(kvg, ea2, e.reshape(E, 1), fd['we1'],
                                     fd['B'], fd['cm'])
        mshe = jnp.concatenate([m0, m1, m2, m3], axis=1)
        agg = jax.ops.segment_sum(mshe, col, num_segments=N)
        if li + 1 < len(folded):
            h, tq, tkv = _upd_kernel(h, agg, s, fd['M2'], fd['b2m'],
                                     folded[li + 1]['F'])

    fd = folded[-1]
    hsum, counts = _fin_kernel(h, agg, s, fd['M2'], fd['b2m'], oh)
    h_graph = hsum / jnp.maximum(counts.reshape(8, 1), 1.0)

    preds = []
    for name in ['neff', 'Aeff', 'NL', 'Disp', 'GVD']:
        z = h_graph
        layers = params['heads'][name]
        for i, (W, b) in enumerate(layers):
            z = z @ W + b
            if i < len(layers) - 1:
                z = jax.nn.silu(z)
        preds.append(z)
    return jnp.concatenate(preds, axis=-1)
